# Initial kernel scaffold; baseline (speedup 1.0000x reference)
import jax
import jax.numpy as jnp
from jax.experimental import pallas as pl


def kernel(flat, cu_seqlens, Wx_f, Wh_f, b_f, Wx_b, Wh_b, b_b, Wa, Wo, Wcx, Wch, bc):
    raise NotImplementedError("write your pallas kernel here")


# speedup vs baseline: 13.2007x; 13.2007x over previous
import jax
import jax.numpy as jnp
from jax.experimental import pallas as pl
from jax.experimental.pallas import tpu as pltpu

B = 16
L = 512
D = 256
H = 256
HC = 512
T = 4096

TB = 32
NL = L // TB


def _dot(a, b):
    return jnp.dot(a, b, preferred_element_type=jnp.float32)


def _pad_kernel(cu_ref, flat_ref, out_ref):
    b = pl.program_id(0)
    start = cu_ref[b]
    seg_len = cu_ref[b + 1] - start
    base = (start // 8) * 8
    rem = start - base
    vals8 = flat_ref[pl.ds(base, L + 8), :]
    vals = pltpu.roll(vals8, (L + 8) - rem, 0)[:L]
    tloc = jax.lax.broadcasted_iota(jnp.int32, (L, 1), 0)
    out_ref[0] = jnp.where(tloc < seg_len, vals, 0.0)


def _pad_call(cu, flat_pad):
    grid_spec = pltpu.PrefetchScalarGridSpec(
        num_scalar_prefetch=1,
        grid=(B,),
        in_specs=[pl.BlockSpec((T + L + 8, D), lambda b, cu_ref: (0, 0))],
        out_specs=pl.BlockSpec((1, L, D), lambda b, cu_ref: (b, 0, 0)),
    )
    return pl.pallas_call(
        _pad_kernel,
        grid_spec=grid_spec,
        out_shape=jax.ShapeDtypeStruct((B, L, D), jnp.float32),
    )(cu, flat_pad)


def _gru_step(h, gx, gh, mask):
    z = jax.nn.sigmoid(gx[:, :H] + gh[:, :H])
    r = jax.nn.sigmoid(gx[:, H:2 * H] + gh[:, H:2 * H])
    n = jnp.tanh(gx[:, 2 * H:] + r * gh[:, 2 * H:])
    h_new = (1.0 - z) * n + z * h
    return jnp.where(mask, h_new, h)


def _scan_kernel(xf_ref, xb_ref, Wxf_ref, Whf_ref, bf_ref,
                 Wxb_ref, Whb_ref, bb_ref, len_ref,
                 ysf_ref, ysb_ref, hf_ref, hb_ref,
                 hf_acc, hb_acc):
    i = pl.program_id(0)

    @pl.when(i == 0)
    def _init():
        hf_acc[...] = jnp.zeros_like(hf_acc)
        hb_acc[...] = jnp.zeros_like(hb_acc)

    lengths = len_ref[:, 0:1]

    gxf = (_dot(xf_ref[...].reshape(B * TB, D), Wxf_ref[...])
           + bf_ref[...]).reshape(B, TB, 3 * H)
    gxb = (_dot(xb_ref[...].reshape(B * TB, D), Wxb_ref[...])
           + bb_ref[...]).reshape(B, TB, 3 * H)

    hf = hf_acc[...]
    hb = hb_acc[...]
    for k in range(TB):
        t_f = i * TB + k
        hf = _gru_step(hf, gxf[:, k, :], _dot(hf, Whf_ref[...]), lengths > t_f)
        ysf_ref[:, k, :] = hf
        kk = TB - 1 - k
        t_b = (NL - 1 - i) * TB + kk
        hb = _gru_step(hb, gxb[:, kk, :], _dot(hb, Whb_ref[...]), lengths > t_b)
        ysb_ref[:, kk, :] = hb
    hf_acc[...] = hf
    hb_acc[...] = hb
    hf_ref[...] = hf
    hb_ref[...] = hb


def _scan_call(padded, Wx_f, Wh_f, b_f, Wx_b, Wh_b, b_b, lengths2d):
    const = lambda i: (0, 0)
    return pl.pallas_call(
        _scan_kernel,
        grid=(NL,),
        in_specs=[
            pl.BlockSpec((B, TB, D), lambda i: (0, i, 0)),
            pl.BlockSpec((B, TB, D), lambda i: (0, NL - 1 - i, 0)),
            pl.BlockSpec((D, 3 * H), const),
            pl.BlockSpec((H, 3 * H), const),
            pl.BlockSpec((1, 3 * H), const),
            pl.BlockSpec((D, 3 * H), const),
            pl.BlockSpec((H, 3 * H), const),
            pl.BlockSpec((1, 3 * H), const),
            pl.BlockSpec((B, 128), const),
        ],
        out_specs=[
            pl.BlockSpec((B, TB, H), lambda i: (0, i, 0)),
            pl.BlockSpec((B, TB, H), lambda i: (0, NL - 1 - i, 0)),
            pl.BlockSpec((B, H), const),
            pl.BlockSpec((B, H), const),
        ],
        out_shape=[
            jax.ShapeDtypeStruct((B, L, H), jnp.float32),
            jax.ShapeDtypeStruct((B, L, H), jnp.float32),
            jax.ShapeDtypeStruct((B, H), jnp.float32),
            jax.ShapeDtypeStruct((B, H), jnp.float32),
        ],
        scratch_shapes=[
            pltpu.VMEM((B, H), jnp.float32),
            pltpu.VMEM((B, H), jnp.float32),
        ],
    )(padded, padded, Wx_f, Wh_f, b_f, Wx_b, Wh_b, b_b, lengths2d)


def _attn_kernel(ysf_ref, ysb_ref, hf_ref, hb_ref, Wa_ref, Wo_ref,
                 Wcx_ref, Wch_ref, bc_ref, len_ref,
                 ctx_ref, pattn_ref):
    hf = hf_ref[...]
    hb = hb_ref[...]
    sent_final = jnp.concatenate([hf, hb], axis=-1)
    q = _dot(sent_final, Wa_ref[...])
    qf = q[:, :H]
    qb = q[:, H:]

    ysf = ysf_ref[...].reshape(B * L, H)
    ysb = ysb_ref[...].reshape(B * L, H)
    sall = (_dot(ysf, qf.T) + _dot(ysb, qb.T)).reshape(B, L, B)
    i0 = jax.lax.broadcasted_iota(jnp.int32, (B, L, B), 0)
    i2 = jax.lax.broadcasted_iota(jnp.int32, (B, L, B), 2)
    s = jnp.sum(jnp.where(i0 == i2, sall, 0.0), axis=-1)
    lengths = len_ref[:, 0:1]
    lidx = jax.lax.broadcasted_iota(jnp.int32, (B, L), 1)
    s = jnp.where(lidx < lengths, s, -1e9)
    m = jnp.max(s, axis=-1, keepdims=True)
    e = jnp.exp(s - m)
    p = e / jnp.sum(e, axis=-1, keepdims=True)
    pattn_ref[...] = p

    j0 = jax.lax.broadcasted_iota(jnp.int32, (B, B, L), 0)
    j1 = jax.lax.broadcasted_iota(jnp.int32, (B, B, L), 1)
    pbd = jnp.where(j0 == j1, p[:, None, :], 0.0).reshape(B, B * L)
    ctxf = _dot(pbd, ysf)
    ctxb = _dot(pbd, ysb)
    cat = jnp.concatenate([ctxf, ctxb, hf, hb], axis=-1)
    sent_vec = jnp.tanh(_dot(cat, Wo_ref[...]))

    gxc = _dot(sent_vec, Wcx_ref[...]) + bc_ref[...]
    h = jnp.zeros((1, HC), jnp.float32)
    for idx in range(B):
        gx = gxc[idx:idx + 1, :]
        gh = _dot(h, Wch_ref[...])
        z = jax.nn.sigmoid(gx[:, :HC] + gh[:, :HC])
        r = jax.nn.sigmoid(gx[:, HC:2 * HC] + gh[:, HC:2 * HC])
        n = jnp.tanh(gx[:, 2 * HC:] + r * gh[:, 2 * HC:])
        h = (1.0 - z) * n + z * h
        ctx_ref[pl.ds(idx, 1), :] = h


def _attn_call(ysf, ysb, hf, hb, Wa, Wo, Wcx, Wch, bc, lengths2d):
    full = lambda shape: pl.BlockSpec(shape, lambda: tuple(0 for _ in shape))
    return pl.pallas_call(
        _attn_kernel,
        in_specs=[
            full((B, L, H)), full((B, L, H)), full((B, H)), full((B, H)),
            full((2 * H, 2 * H)), full((4 * H, HC)),
            full((HC, 3 * HC)), full((HC, 3 * HC)), full((1, 3 * HC)),
            full((B, 128)),
        ],
        out_specs=[full((B, HC)), full((B, L))],
        out_shape=[
            jax.ShapeDtypeStruct((B, HC), jnp.float32),
            jax.ShapeDtypeStruct((B, L), jnp.float32),
        ],
    )(ysf, ysb, hf, hb, Wa, Wo, Wcx, Wch, bc, lengths2d)


def kernel(flat, cu_seqlens, Wx_f, Wh_f, b_f, Wx_b, Wh_b, b_b, Wa, Wo, Wcx, Wch, bc):
    cu = cu_seqlens.astype(jnp.int32)
    flat_pad = jnp.pad(flat, ((0, L + 8), (0, 0)))
    lengths2d = jnp.broadcast_to((cu[1:] - cu[:-1])[:, None], (B, 128))

    padded = _pad_call(cu, flat_pad)
    ysf, ysb, hf, hb = _scan_call(
        padded, Wx_f, Wh_f, b_f.reshape(1, -1),
        Wx_b, Wh_b, b_b.reshape(1, -1), lengths2d)
    ctx_mem, p_attn = _attn_call(
        ysf, ysb, hf, hb, Wa, Wo, Wcx, Wch, bc.reshape(1, -1), lengths2d)
    return ctx_mem, p_attn


# speedup vs baseline: 18.1467x; 1.3747x over previous
import jax
import jax.numpy as jnp
from jax.experimental import pallas as pl
from jax.experimental.pallas import tpu as pltpu

B = 16
L = 512
D = 256
H = 256
HC = 512
T = 4096

TB = 32
NL = L // TB


def _dot(a, b):
    return jnp.dot(a, b, preferred_element_type=jnp.float32)


def _pad_kernel(cu_ref, flat_ref, out_ref):
    b = pl.program_id(0)
    start = cu_ref[b]
    seg_len = cu_ref[b + 1] - start
    base = (start // 8) * 8
    rem = start - base
    vals8 = flat_ref[pl.ds(base, L + 8), :]
    vals = pltpu.roll(vals8, (L + 8) - rem, 0)[:L]
    tloc = jax.lax.broadcasted_iota(jnp.int32, (L, 1), 0)
    out_ref[0] = jnp.where(tloc < seg_len, vals, 0.0)


def _pad_call(cu, flat_pad):
    grid_spec = pltpu.PrefetchScalarGridSpec(
        num_scalar_prefetch=1,
        grid=(B,),
        in_specs=[pl.BlockSpec((T + L + 8, D), lambda b, cu_ref: (0, 0))],
        out_specs=pl.BlockSpec((1, L, D), lambda b, cu_ref: (b, 0, 0)),
    )
    return pl.pallas_call(
        _pad_kernel,
        grid_spec=grid_spec,
        out_shape=jax.ShapeDtypeStruct((B, L, D), jnp.float32),
    )(cu, flat_pad)


def _gru_step(h, gx, gh, mask):
    z = jax.nn.sigmoid(gx[:, :H] + gh[:, :H])
    r = jax.nn.sigmoid(gx[:, H:2 * H] + gh[:, H:2 * H])
    n = jnp.tanh(gx[:, 2 * H:] + r * gh[:, 2 * H:])
    h_new = (1.0 - z) * n + z * h
    return jnp.where(mask, h_new, h)


def _scan_kernel(nb_ref, xf_ref, xb_ref, Wxf_ref, Whf_ref, bf_ref,
                 Wxb_ref, Whb_ref, bb_ref, len_ref,
                 ysf_ref, ysb_ref, hf_ref, hb_ref,
                 hf_acc, hb_acc):
    i = pl.program_id(0)
    nb = nb_ref[0]

    @pl.when(i == 0)
    def _init():
        hf_acc[...] = jnp.zeros_like(hf_acc)
        hb_acc[...] = jnp.zeros_like(hb_acc)

    @pl.when(i < nb)
    def _body():
        lengths = len_ref[:, 0:1]

        gxf = (_dot(xf_ref[...].reshape(B * TB, D), Wxf_ref[...])
               + bf_ref[...]).reshape(B, TB, 3 * H)
        gxb = (_dot(xb_ref[...].reshape(B * TB, D), Wxb_ref[...])
               + bb_ref[...]).reshape(B, TB, 3 * H)

        t0_f = i * TB
        t0_b = (nb - 1 - i) * TB
        hf = hf_acc[...]
        hb = hb_acc[...]
        for k in range(TB):
            hf = _gru_step(hf, gxf[:, k, :], _dot(hf, Whf_ref[...]),
                           lengths > t0_f + k)
            ysf_ref[:, k, :] = hf
            kk = TB - 1 - k
            hb = _gru_step(hb, gxb[:, kk, :], _dot(hb, Whb_ref[...]),
                           lengths > t0_b + kk)
            ysb_ref[:, kk, :] = hb
        hf_acc[...] = hf
        hb_acc[...] = hb
        hf_ref[...] = hf
        hb_ref[...] = hb


def _scan_call(nb, padded, Wx_f, Wh_f, b_f, Wx_b, Wh_b, b_b, lengths2d):
    const = lambda i, s: (0, 0)
    fwd = lambda i, s: (0, jnp.minimum(i, s[0] - 1), 0)
    bwd = lambda i, s: (0, jnp.maximum(s[0] - 1 - i, 0), 0)
    grid_spec = pltpu.PrefetchScalarGridSpec(
        num_scalar_prefetch=1,
        grid=(NL,),
        in_specs=[
            pl.BlockSpec((B, TB, D), fwd),
            pl.BlockSpec((B, TB, D), bwd),
            pl.BlockSpec((D, 3 * H), const),
            pl.BlockSpec((H, 3 * H), const),
            pl.BlockSpec((1, 3 * H), const),
            pl.BlockSpec((D, 3 * H), const),
            pl.BlockSpec((H, 3 * H), const),
            pl.BlockSpec((1, 3 * H), const),
            pl.BlockSpec((B, 128), const),
        ],
        out_specs=[
            pl.BlockSpec((B, TB, H), fwd),
            pl.BlockSpec((B, TB, H), bwd),
            pl.BlockSpec((B, H), const),
            pl.BlockSpec((B, H), const),
        ],
        scratch_shapes=[
            pltpu.VMEM((B, H), jnp.float32),
            pltpu.VMEM((B, H), jnp.float32),
        ],
    )
    return pl.pallas_call(
        _scan_kernel,
        grid_spec=grid_spec,
        out_shape=[
            jax.ShapeDtypeStruct((B, L, H), jnp.float32),
            jax.ShapeDtypeStruct((B, L, H), jnp.float32),
            jax.ShapeDtypeStruct((B, H), jnp.float32),
            jax.ShapeDtypeStruct((B, H), jnp.float32),
        ],
    )(nb, padded, padded, Wx_f, Wh_f, b_f, Wx_b, Wh_b, b_b, lengths2d)


def _attn_kernel(ysf_ref, ysb_ref, hf_ref, hb_ref, Wa_ref, Wo_ref,
                 Wcx_ref, Wch_ref, bc_ref, len_ref,
                 ctx_ref, pattn_ref):
    hf = hf_ref[...]
    hb = hb_ref[...]
    sent_final = jnp.concatenate([hf, hb], axis=-1)
    q = _dot(sent_final, Wa_ref[...])
    qf = q[:, :H]
    qb = q[:, H:]

    lengths3 = len_ref[:, 0:1].reshape(B, 1, 1)
    li3 = jax.lax.broadcasted_iota(jnp.int32, (B, L, 1), 1)
    valid3 = li3 < lengths3
    ysf = jnp.where(valid3, ysf_ref[...], 0.0).reshape(B * L, H)
    ysb = jnp.where(valid3, ysb_ref[...], 0.0).reshape(B * L, H)
    sall = (_dot(ysf, qf.T) + _dot(ysb, qb.T)).reshape(B, L, B)
    i0 = jax.lax.broadcasted_iota(jnp.int32, (B, L, B), 0)
    i2 = jax.lax.broadcasted_iota(jnp.int32, (B, L, B), 2)
    s = jnp.sum(jnp.where(i0 == i2, sall, 0.0), axis=-1)
    lengths = len_ref[:, 0:1]
    lidx = jax.lax.broadcasted_iota(jnp.int32, (B, L), 1)
    s = jnp.where(lidx < lengths, s, -1e9)
    m = jnp.max(s, axis=-1, keepdims=True)
    e = jnp.exp(s - m)
    p = e / jnp.sum(e, axis=-1, keepdims=True)
    pattn_ref[...] = p

    j0 = jax.lax.broadcasted_iota(jnp.int32, (B, B, L), 0)
    j1 = jax.lax.broadcasted_iota(jnp.int32, (B, B, L), 1)
    pbd = jnp.where(j0 == j1, p[:, None, :], 0.0).reshape(B, B * L)
    ctxf = _dot(pbd, ysf)
    ctxb = _dot(pbd, ysb)
    cat = jnp.concatenate([ctxf, ctxb, hf, hb], axis=-1)
    sent_vec = jnp.tanh(_dot(cat, Wo_ref[...]))

    gxc = _dot(sent_vec, Wcx_ref[...]) + bc_ref[...]
    h = jnp.zeros((1, HC), jnp.float32)
    for idx in range(B):
        gx = gxc[idx:idx + 1, :]
        gh = _dot(h, Wch_ref[...])
        z = jax.nn.sigmoid(gx[:, :HC] + gh[:, :HC])
        r = jax.nn.sigmoid(gx[:, HC:2 * HC] + gh[:, HC:2 * HC])
        n = jnp.tanh(gx[:, 2 * HC:] + r * gh[:, 2 * HC:])
        h = (1.0 - z) * n + z * h
        ctx_ref[pl.ds(idx, 1), :] = h


def _attn_call(ysf, ysb, hf, hb, Wa, Wo, Wcx, Wch, bc, lengths2d):
    full = lambda shape: pl.BlockSpec(shape, lambda: tuple(0 for _ in shape))
    return pl.pallas_call(
        _attn_kernel,
        in_specs=[
            full((B, L, H)), full((B, L, H)), full((B, H)), full((B, H)),
            full((2 * H, 2 * H)), full((4 * H, HC)),
            full((HC, 3 * HC)), full((HC, 3 * HC)), full((1, 3 * HC)),
            full((B, 128)),
        ],
        out_specs=[full((B, HC)), full((B, L))],
        out_shape=[
            jax.ShapeDtypeStruct((B, HC), jnp.float32),
            jax.ShapeDtypeStruct((B, L), jnp.float32),
        ],
    )(ysf, ysb, hf, hb, Wa, Wo, Wcx, Wch, bc, lengths2d)


def kernel(flat, cu_seqlens, Wx_f, Wh_f, b_f, Wx_b, Wh_b, b_b, Wa, Wo, Wcx, Wch, bc):
    cu = cu_seqlens.astype(jnp.int32)
    flat_pad = jnp.pad(flat, ((0, L + 8), (0, 0)))
    lengths = cu[1:] - cu[:-1]
    lengths2d = jnp.broadcast_to(lengths[:, None], (B, 128))
    nb = ((jnp.max(lengths) + TB - 1) // TB).reshape(1).astype(jnp.int32)

    padded = _pad_call(cu, flat_pad)
    ysf, ysb, hf, hb = _scan_call(
        nb, padded, Wx_f, Wh_f, b_f.reshape(1, -1),
        Wx_b, Wh_b, b_b.reshape(1, -1), lengths2d)
    ctx_mem, p_attn = _attn_call(
        ysf, ysb, hf, hb, Wa, Wo, Wcx, Wch, bc.reshape(1, -1), lengths2d)
    return ctx_mem, p_attn


# speedup vs baseline: 21.0705x; 1.1611x over previous
import jax
import jax.numpy as jnp
from jax.experimental import pallas as pl
from jax.experimental.pallas import tpu as pltpu

B = 16
L = 512
D = 256
H = 256
HC = 512
T = 4096

TB = 32
NL = L // TB


def _dot(a, b):
    return jnp.dot(a.astype(jnp.bfloat16), b.astype(jnp.bfloat16),
                   preferred_element_type=jnp.float32)


def _pad_kernel(cu_ref, flat_ref, out_ref):
    for b in range(B):
        start = cu_ref[b]
        seg_len = cu_ref[b + 1] - start
        base = (start // 8) * 8
        rem = start - base
        vals8 = flat_ref[pl.ds(base, L + 8), :]
        vals = pltpu.roll(vals8, (L + 8) - rem, 0)[:L]
        tloc = jax.lax.broadcasted_iota(jnp.int32, (L, 1), 0)
        out_ref[:, b, :] = jnp.where(tloc < seg_len, vals, 0.0)


def _pad_call(cu, flat_pad):
    grid_spec = pltpu.PrefetchScalarGridSpec(
        num_scalar_prefetch=1,
        grid=(1,),
        in_specs=[pl.BlockSpec((T + L + 8, D), lambda i, s: (0, 0))],
        out_specs=pl.BlockSpec((L, B, D), lambda i, s: (0, 0, 0)),
    )
    return pl.pallas_call(
        _pad_kernel,
        grid_spec=grid_spec,
        out_shape=jax.ShapeDtypeStruct((L, B, D), jnp.float32),
    )(cu, flat_pad)


def _gru_step(h, gx, gh, mask):
    z = jax.nn.sigmoid(gx[:, :H] + gh[:, :H])
    r = jax.nn.sigmoid(gx[:, H:2 * H] + gh[:, H:2 * H])
    n = jnp.tanh(gx[:, 2 * H:] + r * gh[:, 2 * H:])
    h_new = (1.0 - z) * n + z * h
    return jnp.where(mask, h_new, h)


def _scan_kernel(nb_ref, xf_ref, xb_ref, Wxf_ref, Whf_ref, bf_ref,
                 Wxb_ref, Whb_ref, bb_ref, len_ref,
                 ysf_ref, ysb_ref, hf_ref, hb_ref,
                 hf_acc, hb_acc):
    i = pl.program_id(0)
    nb = nb_ref[0]

    @pl.when(i == 0)
    def _init():
        hf_acc[...] = jnp.zeros_like(hf_acc)
        hb_acc[...] = jnp.zeros_like(hb_acc)

    @pl.when(i < nb)
    def _body():
        lengths = len_ref[:, 0:1]

        gxf = _dot(xf_ref[...].reshape(TB * B, D), Wxf_ref[...]) + bf_ref[...]
        gxb = _dot(xb_ref[...].reshape(TB * B, D), Wxb_ref[...]) + bb_ref[...]

        t0_f = i * TB
        t0_b = (nb - 1 - i) * TB
        hf = hf_acc[...]
        hb = hb_acc[...]
        for k in range(TB):
            hf = _gru_step(hf, gxf[k * B:(k + 1) * B], _dot(hf, Whf_ref[...]),
                           lengths > t0_f + k)
            ysf_ref[k] = hf
            kk = TB - 1 - k
            hb = _gru_step(hb, gxb[kk * B:(kk + 1) * B], _dot(hb, Whb_ref[...]),
                           lengths > t0_b + kk)
            ysb_ref[kk] = hb
        hf_acc[...] = hf
        hb_acc[...] = hb
        hf_ref[...] = hf
        hb_ref[...] = hb


def _scan_call(nb, padded, Wx_f, Wh_f, b_f, Wx_b, Wh_b, b_b, lengths2d):
    const = lambda i, s: (0, 0)
    fwd = lambda i, s: (jnp.minimum(i, s[0] - 1), 0, 0)
    bwd = lambda i, s: (jnp.maximum(s[0] - 1 - i, 0), 0, 0)
    grid_spec = pltpu.PrefetchScalarGridSpec(
        num_scalar_prefetch=1,
        grid=(NL,),
        in_specs=[
            pl.BlockSpec((TB, B, D), fwd),
            pl.BlockSpec((TB, B, D), bwd),
            pl.BlockSpec((D, 3 * H), const),
            pl.BlockSpec((H, 3 * H), const),
            pl.BlockSpec((1, 3 * H), const),
            pl.BlockSpec((D, 3 * H), const),
            pl.BlockSpec((H, 3 * H), const),
            pl.BlockSpec((1, 3 * H), const),
            pl.BlockSpec((B, 128), const),
        ],
        out_specs=[
            pl.BlockSpec((TB, B, H), fwd),
            pl.BlockSpec((TB, B, H), bwd),
            pl.BlockSpec((B, H), const),
            pl.BlockSpec((B, H), const),
        ],
        scratch_shapes=[
            pltpu.VMEM((B, H), jnp.float32),
            pltpu.VMEM((B, H), jnp.float32),
        ],
    )
    return pl.pallas_call(
        _scan_kernel,
        grid_spec=grid_spec,
        out_shape=[
            jax.ShapeDtypeStruct((L, B, H), jnp.float32),
            jax.ShapeDtypeStruct((L, B, H), jnp.float32),
            jax.ShapeDtypeStruct((B, H), jnp.float32),
            jax.ShapeDtypeStruct((B, H), jnp.float32),
        ],
    )(nb, padded, padded, Wx_f, Wh_f, b_f, Wx_b, Wh_b, b_b, lengths2d)


def _attn_kernel(ysf_ref, ysb_ref, hf_ref, hb_ref, Wa_ref, Wo_ref,
                 Wcx_ref, Wch_ref, bc_ref, len_ref,
                 ctx_ref, pattn_ref):
    hf = hf_ref[...]
    hb = hb_ref[...]
    sent_final = jnp.concatenate([hf, hb], axis=-1)
    q = _dot(sent_final, Wa_ref[...])
    qf = q[:, :H]
    qb = q[:, H:]

    lengths = len_ref[:, 0:1]
    lengths3 = lengths.reshape(1, B, 1)
    li3 = jax.lax.broadcasted_iota(jnp.int32, (L, B, 1), 0)
    valid3 = li3 < lengths3
    ysf = jnp.where(valid3, ysf_ref[...], 0.0).reshape(L * B, H)
    ysb = jnp.where(valid3, ysb_ref[...], 0.0).reshape(L * B, H)

    sall = (_dot(ysf, qf.T) + _dot(ysb, qb.T)).reshape(L, B, B)
    i1 = jax.lax.broadcasted_iota(jnp.int32, (L, B, B), 1)
    i2 = jax.lax.broadcasted_iota(jnp.int32, (L, B, B), 2)
    s = jnp.sum(jnp.where(i1 == i2, sall, 0.0), axis=-1).T
    lidx = jax.lax.broadcasted_iota(jnp.int32, (B, L), 1)
    s = jnp.where(lidx < lengths, s, -1e9)
    m = jnp.max(s, axis=-1, keepdims=True)
    e = jnp.exp(s - m)
    p = e / jnp.sum(e, axis=-1, keepdims=True)
    pattn_ref[...] = p

    j0 = jax.lax.broadcasted_iota(jnp.int32, (B, L, B), 0)
    j2 = jax.lax.broadcasted_iota(jnp.int32, (B, L, B), 2)
    pbd = jnp.where(j0 == j2, p[:, :, None], 0.0).reshape(B, L * B)
    ctxf = _dot(pbd, ysf)
    ctxb = _dot(pbd, ysb)
    cat = jnp.concatenate([ctxf, ctxb, hf, hb], axis=-1)
    sent_vec = jnp.tanh(_dot(cat, Wo_ref[...]))

    gxc = _dot(sent_vec, Wcx_ref[...]) + bc_ref[...]
    h = jnp.zeros((1, HC), jnp.float32)
    for idx in range(B):
        gx = gxc[idx:idx + 1, :]
        gh = _dot(h, Wch_ref[...])
        z = jax.nn.sigmoid(gx[:, :HC] + gh[:, :HC])
        r = jax.nn.sigmoid(gx[:, HC:2 * HC] + gh[:, HC:2 * HC])
        n = jnp.tanh(gx[:, 2 * HC:] + r * gh[:, 2 * HC:])
        h = (1.0 - z) * n + z * h
        ctx_ref[pl.ds(idx, 1), :] = h


def _attn_call(ysf, ysb, hf, hb, Wa, Wo, Wcx, Wch, bc, lengths2d):
    full = lambda shape: pl.BlockSpec(shape, lambda: tuple(0 for _ in shape))
    return pl.pallas_call(
        _attn_kernel,
        in_specs=[
            full((L, B, H)), full((L, B, H)), full((B, H)), full((B, H)),
            full((2 * H, 2 * H)), full((4 * H, HC)),
            full((HC, 3 * HC)), full((HC, 3 * HC)), full((1, 3 * HC)),
            full((B, 128)),
        ],
        out_specs=[full((B, HC)), full((B, L))],
        out_shape=[
            jax.ShapeDtypeStruct((B, HC), jnp.float32),
            jax.ShapeDtypeStruct((B, L), jnp.float32),
        ],
    )(ysf, ysb, hf, hb, Wa, Wo, Wcx, Wch, bc, lengths2d)


def kernel(flat, cu_seqlens, Wx_f, Wh_f, b_f, Wx_b, Wh_b, b_b, Wa, Wo, Wcx, Wch, bc):
    cu = cu_seqlens.astype(jnp.int32)
    flat_pad = jnp.pad(flat, ((0, L + 8), (0, 0)))
    lengths = cu[1:] - cu[:-1]
    lengths2d = jnp.broadcast_to(lengths[:, None], (B, 128))
    nb = ((jnp.max(lengths) + TB - 1) // TB).reshape(1).astype(jnp.int32)
    bf16 = jnp.bfloat16

    padded = _pad_call(cu, flat_pad)
    ysf, ysb, hf, hb = _scan_call(
        nb, padded, Wx_f.astype(bf16), Wh_f.astype(bf16), b_f.reshape(1, -1),
        Wx_b.astype(bf16), Wh_b.astype(bf16), b_b.reshape(1, -1), lengths2d)
    ctx_mem, p_attn = _attn_call(
        ysf, ysb, hf, hb, Wa.astype(bf16), Wo.astype(bf16),
        Wcx.astype(bf16), Wch.astype(bf16), bc.reshape(1, -1), lengths2d)
    return ctx_mem, p_attn
